# baseline (speedup 1.0000x reference)
import functools

import jax
import jax.numpy as jnp
from jax import lax
from jax.experimental import pallas as pl
from jax.experimental.pallas import tpu as pltpu
from jax.experimental.pallas import tpu_sc as plsc

NEI = 8
D = 64
REL = NEI * 2
LAT = NEI * D
SUB = LAT // 16
ROWS = SUB + 1
OROW = 40
L = 16


def kernel(idx_flat, latents, idxcache, relpos_cache):
    B = idx_flat.shape[0]
    grid = idxcache.shape[0]
    gtiles = grid // 128
    lat16 = latents.reshape(latents.shape[0] * (D // 16), 16)
    cache_nat = idxcache.reshape(gtiles, 128, NEI).transpose(0, 2, 1).reshape(grid * NEI)
    rel_nat = relpos_cache.reshape(gtiles, 128, NEI, 2).transpose(2, 0, 3, 1).reshape(grid * REL)

    info = plsc.get_sparse_core_info()
    nw = info.num_cores * info.num_subcores
    per_w = B // nw
    C = 64
    n_chunks = per_w // C
    CR = C * SUB

    mesh = plsc.VectorSubcoreMesh(core_axis_name="c", subcore_axis_name="s")

    @functools.partial(
        pl.kernel,
        mesh=mesh,
        compiler_params=pltpu.CompilerParams(
            needs_layout_passes=False, use_tc_tiling_on_sc=False),
        out_type=jax.ShapeDtypeStruct((B * OROW, 16), jnp.float32),
        scratch_types=[
            pltpu.VMEM((per_w,), jnp.int32),
            pltpu.VMEM((per_w * NEI,), jnp.int32),
            pltpu.VMEM((per_w * NEI,), jnp.int32),
            pltpu.VMEM((per_w * REL,), jnp.int32),
            pltpu.VMEM((per_w * REL,), jnp.float32),
            pltpu.VMEM((2, CR), jnp.int32),
            pltpu.VMEM((2, CR, 16), jnp.float32),
            pltpu.VMEM((CR,), jnp.int32),
            pltpu.VMEM((per_w,), jnp.int32),
            pltpu.VMEM((per_w, REL), jnp.float32),
            pltpu.SemaphoreType.DMA,
            pltpu.SemaphoreType.DMA,
            pltpu.SemaphoreType.DMA,
            pltpu.SemaphoreType.DMA,
            pltpu.SemaphoreType.DMA,
            pltpu.SemaphoreType.DMA,
            pltpu.SemaphoreType.DMA,
        ],
    )
    def run(idx_hbm, lat_hbm, cache_hbm, rel_hbm, out_hbm,
            idx_v, fidx_v, nbr_v, ridx16_v, rel1_v, lidx_v, lat_v, opat_v,
            ridx_v, rel_v, sem_c, sem_r, sem_l0, sem_l1, sem_s0, sem_s1,
            sem_rs):
        wid = lax.axis_index("s") * info.num_cores + lax.axis_index("c")
        base = wid * per_w
        pltpu.sync_copy(idx_hbm.at[pl.ds(base, per_w)], idx_v)
        iota = lax.iota(jnp.int32, L)

        @pl.loop(0, per_w // L)
        def _fidx(k):
            g = idx_v[pl.ds(k * L, L)]
            t = lax.shift_right_logical(g, 7)
            m = lax.bitwise_and(g, 127)
            cbase = lax.shift_left(t, 10) + m
            pos = iota * NEI + k * L * NEI
            for j in range(NEI):
                plsc.store_scatter(fidx_v, [pos + j], cbase + j * 128)
        cache_cp = pltpu.async_copy(cache_hbm.at[fidx_v], nbr_v, sem_c)

        @pl.loop(0, per_w // L)
        def _ridx(k):
            g = idx_v[pl.ds(k * L, L)]
            t = lax.shift_right_logical(g, 7)
            m = lax.bitwise_and(g, 127)
            rbase = lax.shift_left(t, 8) + m
            rpos = iota * REL + k * L * REL
            for j in range(NEI):
                for k2 in range(2):
                    plsc.store_scatter(
                        ridx16_v, [rpos + (j * 2 + k2)],
                        rbase + (j * grid * 2 + k2 * 128))
        cache_cp.wait()
        rel_cp = pltpu.async_copy(rel_hbm.at[ridx16_v], rel1_v, sem_r)

        @pl.loop(0, CR // L)
        def _opat(k):
            e = lax.shift_right_logical(k, 1)
            s0 = lax.shift_left(lax.bitwise_and(k, 1), 4)
            opat_v[pl.ds(k * L, L)] = iota + (e * OROW + s0)

        sems_l = (sem_l0, sem_l1)
        sems_s = (sem_s0, sem_s1)

        def stage(ci, p):
            @pl.loop(0, C * NEI // L)
            def _lidx(k):
                n4 = nbr_v[pl.ds(ci * C * NEI + k * L, L)] * 4
                pos = iota * 4 + k * L * 4
                for j in range(4):
                    plsc.store_scatter(lidx_v.at[p], [pos + j], n4 + j)
            return pltpu.async_copy(
                lat_hbm.at[lidx_v.at[p]], lat_v.at[p], sems_l[p])

        def scat(ci, p):
            row0 = base + ci * C
            return pltpu.async_copy(
                lat_v.at[p],
                out_hbm.at[pl.ds(row0 * OROW, C * OROW)].at[opat_v],
                sems_s[p])

        lat_cp0 = stage(0, 0)

        def pair(u, lat_cp0):
            ci0 = u * 2
            lat_cp1 = stage(ci0 + 1, 1)
            lat_cp0.wait()
            st0 = scat(ci0, 0)
            nxt = None
            if u < n_chunks // 2 - 1:
                st0.wait()
                nxt = stage(ci0 + 2, 0)
            lat_cp1.wait()
            st1 = scat(ci0 + 1, 1)
            if u >= n_chunks // 2 - 1:
                st0.wait()
            st1.wait()
            return nxt

        for u in range(n_chunks // 2):
            lat_cp0 = pair(u, lat_cp0)

        rel_cp.wait()

        @pl.loop(0, per_w)
        def _repack(e):
            rel_v[e, :] = rel1_v[pl.ds(e * REL, REL)]

        @pl.loop(0, per_w // L)
        def _ridxout(k):
            ridx_v[pl.ds(k * L, L)] = (iota + (base + k * L)) * OROW + SUB
        pltpu.async_copy(rel_v, out_hbm.at[ridx_v], sem_rs).wait()

    out = run(idx_flat, lat16, cache_nat, rel_nat)

    bt = B // 128
    x = out.reshape(B * OROW * 16 // 128, 128)

    def tbody(x_ref, z_ref):
        x3 = x_ref[...].reshape(128, 5, 128)
        for j in range(4):
            z_ref[j * 16:(j + 1) * 16, 0, :, :] = (
                x3[:, j, :].T.reshape(16, 8, 128))
        z_ref[64:66, 0, :, :] = x3[:, 4, 0:16].T.reshape(2, 8, 128)

    z = pl.pallas_call(
        tbody,
        grid=(bt,),
        in_specs=[pl.BlockSpec((OROW * 16, 128), lambda i: (i, 0))],
        out_specs=pl.BlockSpec((ROWS * 2, 1, 8, 128), lambda i: (0, i, 0, 0)),
        out_shape=jax.ShapeDtypeStruct((ROWS * 2, bt, 8, 128), jnp.float32),
    )(x)
    return z.transpose(1, 3, 0, 2).reshape(B, ROWS * 16)


# speedup vs baseline: 1.1665x; 1.1665x over previous
import functools

import jax
import jax.numpy as jnp
from jax import lax
from jax.experimental import pallas as pl
from jax.experimental.pallas import tpu as pltpu
from jax.experimental.pallas import tpu_sc as plsc

NEI = 8
D = 64
REL = NEI * 2
LAT = NEI * D
SUB = LAT // 16
ROWS = SUB + 1
L = 16


def kernel(idx_flat, latents, idxcache, relpos_cache):
    B = idx_flat.shape[0]
    grid = idxcache.shape[0]
    gtiles = grid // 128
    lat16 = latents.reshape(latents.shape[0] * (D // 16), 16)
    cache_nat = idxcache.reshape(gtiles, 128, NEI).transpose(0, 2, 1).reshape(grid * NEI)
    rel_nat = relpos_cache.reshape(gtiles, 128, NEI, 2).transpose(2, 0, 3, 1).reshape(grid * REL)

    info = plsc.get_sparse_core_info()
    nw = info.num_cores * info.num_subcores
    per_w = B // nw
    C = 64
    n_chunks = per_w // C
    CR = C * SUB

    mesh = plsc.VectorSubcoreMesh(core_axis_name="c", subcore_axis_name="s")

    @functools.partial(
        pl.kernel,
        mesh=mesh,
        compiler_params=pltpu.CompilerParams(
            needs_layout_passes=False, use_tc_tiling_on_sc=False),
        out_type=(jax.ShapeDtypeStruct((B * NEI,), jnp.int32),
                  jax.ShapeDtypeStruct((B, REL), jnp.float32)),
        scratch_types=[
            pltpu.VMEM((per_w,), jnp.int32),
            pltpu.VMEM((per_w * NEI,), jnp.int32),
            pltpu.VMEM((per_w * NEI,), jnp.int32),
            pltpu.VMEM((per_w * REL,), jnp.int32),
            pltpu.VMEM((per_w * REL,), jnp.float32),
            pltpu.VMEM((per_w, REL), jnp.float32),
            pltpu.SemaphoreType.DMA,
            pltpu.SemaphoreType.DMA,
        ],
    )
    def stage_a(idx_hbm, cache_hbm, rel_hbm, nbr_hbm, relrow_hbm,
                idx_v, fidx_v, nbr_v, ridx16_v, rel1_v, rel_v, sem_c, sem_r):
        wid = lax.axis_index("s") * info.num_cores + lax.axis_index("c")
        base = wid * per_w
        pltpu.sync_copy(idx_hbm.at[pl.ds(base, per_w)], idx_v)
        iota = lax.iota(jnp.int32, L)

        @pl.loop(0, per_w // L)
        def _fidx(k):
            g = idx_v[pl.ds(k * L, L)]
            t = lax.shift_right_logical(g, 7)
            m = lax.bitwise_and(g, 127)
            cbase = lax.shift_left(t, 10) + m
            pos = iota * NEI + k * L * NEI
            for j in range(NEI):
                plsc.store_scatter(fidx_v, [pos + j], cbase + j * 128)
        cache_cp = pltpu.async_copy(cache_hbm.at[fidx_v], nbr_v, sem_c)

        @pl.loop(0, per_w // L)
        def _ridx(k):
            g = idx_v[pl.ds(k * L, L)]
            t = lax.shift_right_logical(g, 7)
            m = lax.bitwise_and(g, 127)
            rbase = lax.shift_left(t, 8) + m
            rpos = iota * REL + k * L * REL
            for j in range(NEI):
                for k2 in range(2):
                    plsc.store_scatter(
                        ridx16_v, [rpos + (j * 2 + k2)],
                        rbase + (j * grid * 2 + k2 * 128))
        rel_cp = pltpu.async_copy(rel_hbm.at[ridx16_v], rel1_v, sem_r)
        cache_cp.wait()
        pltpu.sync_copy(nbr_v, nbr_hbm.at[pl.ds(base * NEI, per_w * NEI)])
        rel_cp.wait()

        @pl.loop(0, per_w)
        def _repack(e):
            rel_v[e, :] = rel1_v[pl.ds(e * REL, REL)]
        pltpu.sync_copy(rel_v, relrow_hbm.at[pl.ds(base, per_w)])

    @functools.partial(
        pl.kernel,
        mesh=mesh,
        compiler_params=pltpu.CompilerParams(
            needs_layout_passes=False, use_tc_tiling_on_sc=False),
        out_type=jax.ShapeDtypeStruct((B * ROWS, 16), jnp.float32),
        scratch_types=[
            pltpu.VMEM((per_w * NEI,), jnp.int32),
            pltpu.VMEM((per_w, REL), jnp.float32),
            pltpu.VMEM((2, CR), jnp.int32),
            pltpu.VMEM((2, CR, 16), jnp.float32),
            pltpu.VMEM((CR,), jnp.int32),
            pltpu.VMEM((per_w,), jnp.int32),
            pltpu.SemaphoreType.DMA,
            pltpu.SemaphoreType.DMA,
            pltpu.SemaphoreType.DMA,
            pltpu.SemaphoreType.DMA,
            pltpu.SemaphoreType.DMA,
        ],
    )
    def stage_b(nbr_hbm, relrow_hbm, lat_hbm, out_hbm,
                nbr_v, rel_v, lidx_v, lat_v, opat_v, ridx_v,
                sem_l0, sem_l1, sem_s0, sem_s1, sem_rs):
        wid = lax.axis_index("s") * info.num_cores + lax.axis_index("c")
        base = wid * per_w
        iota = lax.iota(jnp.int32, L)
        pltpu.sync_copy(nbr_hbm.at[pl.ds(base * NEI, per_w * NEI)], nbr_v)
        rel_in = pltpu.async_copy(
            relrow_hbm.at[pl.ds(base, per_w)], rel_v, sem_rs)

        @pl.loop(0, CR // L)
        def _opat(k):
            e = lax.shift_right_logical(k, 1)
            s0 = lax.shift_left(lax.bitwise_and(k, 1), 4)
            opat_v[pl.ds(k * L, L)] = iota + (e * ROWS + s0)

        sems_l = (sem_l0, sem_l1)
        sems_s = (sem_s0, sem_s1)

        def stage(ci, p):
            @pl.loop(0, C * NEI // L)
            def _lidx(k):
                n4 = nbr_v[pl.ds(ci * C * NEI + k * L, L)] * 4
                pos = iota * 4 + k * L * 4
                for j in range(4):
                    plsc.store_scatter(lidx_v.at[p], [pos + j], n4 + j)
            return pltpu.async_copy(
                lat_hbm.at[lidx_v.at[p]], lat_v.at[p], sems_l[p])

        def scat(ci, p):
            row0 = base + ci * C
            return pltpu.async_copy(
                lat_v.at[p],
                out_hbm.at[pl.ds(row0 * ROWS, C * ROWS)].at[opat_v],
                sems_s[p])

        lat_cp0 = stage(0, 0)

        def pair(u, lat_cp0):
            ci0 = u * 2
            lat_cp1 = stage(ci0 + 1, 1)
            lat_cp0.wait()
            st0 = scat(ci0, 0)
            nxt = None
            if u < n_chunks // 2 - 1:
                st0.wait()
                nxt = stage(ci0 + 2, 0)
            lat_cp1.wait()
            st1 = scat(ci0 + 1, 1)
            if u >= n_chunks // 2 - 1:
                st0.wait()
            st1.wait()
            return nxt

        for u in range(n_chunks // 2):
            lat_cp0 = pair(u, lat_cp0)

        rel_in.wait()

        @pl.loop(0, per_w // L)
        def _ridxout(k):
            ridx_v[pl.ds(k * L, L)] = (iota + (base + k * L)) * ROWS + SUB
        pltpu.async_copy(rel_v, out_hbm.at[ridx_v], sem_rs).wait()

    nbr_all, rel_all = stage_a(idx_flat, cache_nat, rel_nat)
    out = stage_b(nbr_all, rel_all, lat16)
    return out.reshape(B, ROWS * 16)


# speedup vs baseline: 1.1688x; 1.0020x over previous
import functools

import jax
import jax.numpy as jnp
from jax import lax
from jax.experimental import pallas as pl
from jax.experimental.pallas import tpu as pltpu
from jax.experimental.pallas import tpu_sc as plsc

NEI = 8
D = 64
REL = NEI * 2
LAT = NEI * D
SUB = LAT // 16
ROWS = SUB + 1
L = 16


def kernel(idx_flat, latents, idxcache, relpos_cache):
    B = idx_flat.shape[0]
    grid = idxcache.shape[0]
    gtiles = grid // 128
    lat16 = latents.reshape(latents.shape[0] * (D // 16), 16)
    cache_nat = idxcache.reshape(gtiles, 128, NEI).transpose(0, 2, 1).reshape(grid * NEI)
    rel_nat = relpos_cache.reshape(gtiles, 128, NEI, 2).transpose(2, 0, 3, 1).reshape(grid * REL)

    info = plsc.get_sparse_core_info()
    nw = info.num_cores * info.num_subcores
    per_w = B // nw
    C = 64
    n_chunks = per_w // C
    CR = C * SUB

    mesh = plsc.VectorSubcoreMesh(core_axis_name="c", subcore_axis_name="s")

    @functools.partial(
        pl.kernel,
        mesh=mesh,
        compiler_params=pltpu.CompilerParams(
            needs_layout_passes=False, use_tc_tiling_on_sc=False),
        out_type=(jax.ShapeDtypeStruct((B * NEI,), jnp.int32),
                  jax.ShapeDtypeStruct((B, REL), jnp.float32)),
        scratch_types=[
            pltpu.VMEM((per_w,), jnp.int32),
            pltpu.VMEM((per_w * NEI,), jnp.int32),
            pltpu.VMEM((per_w * NEI,), jnp.int32),
            pltpu.VMEM((per_w * REL,), jnp.int32),
            pltpu.VMEM((per_w * REL,), jnp.float32),
            pltpu.VMEM((per_w, REL), jnp.float32),
            pltpu.SemaphoreType.DMA,
            pltpu.SemaphoreType.DMA,
        ],
    )
    def stage_a(idx_hbm, cache_hbm, rel_hbm, nbr_hbm, relrow_hbm,
                idx_v, fidx_v, nbr_v, ridx16_v, rel1_v, rel_v, sem_c, sem_r):
        wid = lax.axis_index("s") * info.num_cores + lax.axis_index("c")
        base = wid * per_w
        pltpu.sync_copy(idx_hbm.at[pl.ds(base, per_w)], idx_v)
        iota = lax.iota(jnp.int32, L)

        @pl.loop(0, per_w // L)
        def _fidx(k):
            g = idx_v[pl.ds(k * L, L)]
            t = lax.shift_right_logical(g, 7)
            m = lax.bitwise_and(g, 127)
            cbase = lax.shift_left(t, 10) + m
            pos = iota * NEI + k * L * NEI
            for j in range(NEI):
                plsc.store_scatter(fidx_v, [pos + j], cbase + j * 128)
        cache_cp = pltpu.async_copy(cache_hbm.at[fidx_v], nbr_v, sem_c)

        @pl.loop(0, per_w // L)
        def _ridx(k):
            g = idx_v[pl.ds(k * L, L)]
            t = lax.shift_right_logical(g, 7)
            m = lax.bitwise_and(g, 127)
            rbase = lax.shift_left(t, 8) + m
            rpos = iota * REL + k * L * REL
            for j in range(NEI):
                for k2 in range(2):
                    plsc.store_scatter(
                        ridx16_v, [rpos + (j * 2 + k2)],
                        rbase + (j * grid * 2 + k2 * 128))
        rel_cp = pltpu.async_copy(rel_hbm.at[ridx16_v], rel1_v, sem_r)
        cache_cp.wait()
        pltpu.sync_copy(nbr_v, nbr_hbm.at[pl.ds(base * NEI, per_w * NEI)])
        rel_cp.wait()

        @pl.loop(0, per_w)
        def _repack(e):
            rel_v[e, :] = rel1_v[pl.ds(e * REL, REL)]
        pltpu.sync_copy(rel_v, relrow_hbm.at[pl.ds(base, per_w)])

    @functools.partial(
        pl.kernel,
        mesh=mesh,
        compiler_params=pltpu.CompilerParams(
            needs_layout_passes=False, use_tc_tiling_on_sc=False),
        out_type=jax.ShapeDtypeStruct((B * ROWS, 16), jnp.float32),
        scratch_types=[
            pltpu.VMEM((per_w * NEI,), jnp.int32),
            pltpu.VMEM((per_w, REL), jnp.float32),
            pltpu.VMEM((3, CR), jnp.int32),
            pltpu.VMEM((3, CR, 16), jnp.float32),
            pltpu.VMEM((CR,), jnp.int32),
            pltpu.VMEM((per_w,), jnp.int32),
            pltpu.SemaphoreType.DMA,
            pltpu.SemaphoreType.DMA,
            pltpu.SemaphoreType.DMA,
            pltpu.SemaphoreType.DMA,
            pltpu.SemaphoreType.DMA,
            pltpu.SemaphoreType.DMA,
            pltpu.SemaphoreType.DMA,
        ],
    )
    def stage_b(nbr_hbm, relrow_hbm, lat_hbm, out_hbm,
                nbr_v, rel_v, lidx_v, lat_v, opat_v, ridx_v,
                sem_l0, sem_l1, sem_l2, sem_s0, sem_s1, sem_s2, sem_rs):
        wid = lax.axis_index("s") * info.num_cores + lax.axis_index("c")
        base = wid * per_w
        iota = lax.iota(jnp.int32, L)
        pltpu.sync_copy(nbr_hbm.at[pl.ds(base * NEI, per_w * NEI)], nbr_v)
        rel_in = pltpu.async_copy(
            relrow_hbm.at[pl.ds(base, per_w)], rel_v, sem_rs)

        @pl.loop(0, CR // L)
        def _opat(k):
            e = lax.shift_right_logical(k, 1)
            s0 = lax.shift_left(lax.bitwise_and(k, 1), 4)
            opat_v[pl.ds(k * L, L)] = iota + (e * ROWS + s0)

        sems_l = (sem_l0, sem_l1, sem_l2)
        sems_s = (sem_s0, sem_s1, sem_s2)

        def stage(ci, p):
            @pl.loop(0, C * NEI // L)
            def _lidx(k):
                n4 = nbr_v[pl.ds(ci * C * NEI + k * L, L)] * 4
                pos = iota * 4 + k * L * 4
                for j in range(4):
                    plsc.store_scatter(lidx_v.at[p], [pos + j], n4 + j)
            return pltpu.async_copy(
                lat_hbm.at[lidx_v.at[p]], lat_v.at[p], sems_l[p])

        def scat(ci, p):
            row0 = base + ci * C
            return pltpu.async_copy(
                lat_v.at[p],
                out_hbm.at[pl.ds(row0 * ROWS, C * ROWS)].at[opat_v],
                sems_s[p])

        cps, sts = {}, {}
        for ci in range(n_chunks):
            p = ci % 3
            if ci >= 3:
                sts[ci - 3].wait()
            cps[ci] = stage(ci, p)
            if ci >= 1:
                cps[ci - 1].wait()
                sts[ci - 1] = scat(ci - 1, (ci - 1) % 3)
        cps[n_chunks - 1].wait()
        sts[n_chunks - 1] = scat(n_chunks - 1, (n_chunks - 1) % 3)
        for ci in range(max(0, n_chunks - 3), n_chunks):
            sts[ci].wait()

        rel_in.wait()

        @pl.loop(0, per_w // L)
        def _ridxout(k):
            ridx_v[pl.ds(k * L, L)] = (iota + (base + k * L)) * ROWS + SUB
        pltpu.async_copy(rel_v, out_hbm.at[ridx_v], sem_rs).wait()

    nbr_all, rel_all = stage_a(idx_flat, cache_nat, rel_nat)
    out = stage_b(nbr_all, rel_all, lat16)
    return out.reshape(B, ROWS * 16)
